# baseline (speedup 1.0000x reference)
import jax
import jax.numpy as jnp
from jax import lax
from jax.experimental import pallas as pl
from jax.experimental.pallas import tpu as pltpu
from jax.experimental.pallas import tpu_sc as plsc

NUM_ENTITIES = 1000000
NUM_RELATIONS = 1000
EMBED_DIM = 64
PK = 128
BATCH = 16384

_info = plsc.get_sparse_core_info()
NC, NS, L = _info.num_cores, _info.num_subcores, _info.num_lanes
NW = NC * NS
RPW = BATCH // NW
G = 16
CHUNK = 128
GROUPS_PER_CHUNK = CHUNK // G
N_CHUNKS = RPW // CHUNK
D_VECS = EMBED_DIM // L


def _body(e1_hbm, r_hbm, er2, ei2, relpk, out_r, out_i,
          eidx_v, ridx_v, pe_v, er_v, ei_v, rel_v, or_v, oi_v, sem):
    wid = lax.axis_index("s") * NC + lax.axis_index("c")
    base = wid * RPW
    pltpu.sync_copy(e1_hbm.at[pl.ds(base, RPW)], eidx_v)
    pltpu.sync_copy(r_hbm.at[pl.ds(base, RPW)], ridx_v)

    def pe_body(i, carry):
        sl = pl.ds(i * G, G)
        pe_v[sl] = lax.shift_right_logical(eidx_v[sl], 1)
        return carry

    lax.fori_loop(0, RPW // G, pe_body, 0)

    def chunk_body(ci, carry):
        csl = pl.ds(ci * CHUNK, CHUNK)
        cp1 = pltpu.async_copy(er2.at[pe_v.at[csl]], er_v, sem)
        cp2 = pltpu.async_copy(ei2.at[pe_v.at[csl]], ei_v, sem)
        cp3 = pltpu.async_copy(relpk.at[ridx_v.at[csl]], rel_v, sem)
        cp1.wait()
        cp2.wait()
        cp3.wait()

        def grp_body(g, gcarry):
            grow = ci * GROUPS_PER_CHUNK + g
            e_vec = eidx_v[pl.ds(grow * G, G)]
            row0 = g * G
            for j in range(G):
                off = (e_vec[j] & 1) * EMBED_DIM
                row = row0 + j
                for cb in range(D_VECS):
                    osl = pl.ds(off + cb * L, L)
                    sl = pl.ds(cb * L, L)
                    sli = pl.ds(EMBED_DIM + cb * L, L)
                    a = er_v[row, osl]
                    b = ei_v[row, osl]
                    cc = rel_v[row, sl]
                    d = rel_v[row, sli]
                    or_v[row, sl] = a * cc - b * d
                    oi_v[row, sl] = a * d + b * cc
            return gcarry

        lax.fori_loop(0, GROUPS_PER_CHUNK, grp_body, 0)
        off = base + ci * CHUNK
        pltpu.sync_copy(or_v, out_r.at[pl.ds(off, CHUNK)])
        pltpu.sync_copy(oi_v, out_i.at[pl.ds(off, CHUNK)])
        return carry

    lax.fori_loop(0, N_CHUNKS, chunk_body, 0)


@jax.jit
def kernel(e1, r, ent_real, ent_img, rel_real, rel_img):
    er2 = ent_real.reshape(NUM_ENTITIES // 2, PK)
    ei2 = ent_img.reshape(NUM_ENTITIES // 2, PK)
    relpk = jnp.concatenate([rel_real, rel_img], axis=1)
    mesh = plsc.VectorSubcoreMesh(core_axis_name="c", subcore_axis_name="s")
    out_shape = jax.ShapeDtypeStruct((BATCH, EMBED_DIM), jnp.float32)
    buf = pltpu.VMEM((CHUNK, PK), jnp.float32)
    fn = pl.kernel(
        _body,
        out_type=(out_shape, out_shape),
        mesh=mesh,
        scratch_types=[
            pltpu.VMEM((RPW,), jnp.int32),
            pltpu.VMEM((RPW,), jnp.int32),
            pltpu.VMEM((RPW,), jnp.int32),
            buf, buf, buf,
            pltpu.VMEM((CHUNK, EMBED_DIM), jnp.float32),
            pltpu.VMEM((CHUNK, EMBED_DIM), jnp.float32),
            pltpu.SemaphoreType.DMA,
        ],
        compiler_params=pltpu.CompilerParams(
            use_tc_tiling_on_sc=True, needs_layout_passes=False),
    )
    return fn(e1, r, er2, ei2, relpk)


# speedup vs baseline: 2.2252x; 2.2252x over previous
import jax
import jax.numpy as jnp
from jax import lax
from jax.experimental import pallas as pl
from jax.experimental.pallas import tpu as pltpu
from jax.experimental.pallas import tpu_sc as plsc

NUM_ENTITIES = 1000000
NUM_RELATIONS = 1000
EMBED_DIM = 64
PK = 128
BATCH = 16384

_info = plsc.get_sparse_core_info()
NC, NS, L = _info.num_cores, _info.num_subcores, _info.num_lanes
NW = NC * NS
RPW = BATCH // NW
G = 16
CHUNK = 128
GROUPS_PER_CHUNK = CHUNK // G
N_CHUNKS = RPW // CHUNK
D_VECS = EMBED_DIM // L


def _body(e1_hbm, r_hbm, er3, ei3, relpk, out_r, out_i,
          eidx_v, ridx_v, a_v, b_v, rel_v, or_v, oi_v, sem, rsem):
    wid = lax.axis_index("s") * NC + lax.axis_index("c")
    base = wid * RPW
    pltpu.sync_copy(e1_hbm.at[pl.ds(base, RPW)], eidx_v)
    pltpu.sync_copy(r_hbm.at[pl.ds(base, RPW)], ridx_v)

    def chunk_body(ci, carry):
        pltpu.async_copy(relpk.at[ridx_v.at[pl.ds(ci * CHUNK, CHUNK)]],
                         rel_v, rsem)
        pltpu.make_async_copy(relpk.at[pl.ds(0, CHUNK)], rel_v, rsem).wait()

        def grp_body(g, gcarry):
            grow = ci * GROUPS_PER_CHUNK + g
            e_vec = eidx_v[pl.ds(grow * G, G)]
            for j in range(G):
                pe = e_vec[j] >> 3
                se = e_vec[j] & 7
                pltpu.async_copy(er3.at[pe, se], a_v.at[j], sem)
                pltpu.async_copy(ei3.at[pe, se], b_v.at[j], sem)
            for j in range(G):
                pltpu.make_async_copy(er3.at[0, 0], a_v.at[j], sem).wait()
                pltpu.make_async_copy(er3.at[0, 0], b_v.at[j], sem).wait()
            row0 = g * G
            for j in range(G):
                for cb in range(D_VECS):
                    sl = pl.ds(cb * L, L)
                    sli = pl.ds(EMBED_DIM + cb * L, L)
                    a = a_v[j, sl]
                    b = b_v[j, sl]
                    cc = rel_v[row0 + j, sl]
                    d = rel_v[row0 + j, sli]
                    or_v[row0 + j, sl] = a * cc - b * d
                    oi_v[row0 + j, sl] = a * d + b * cc
            return gcarry

        lax.fori_loop(0, GROUPS_PER_CHUNK, grp_body, 0)
        off = base + ci * CHUNK
        pltpu.sync_copy(or_v, out_r.at[pl.ds(off, CHUNK)])
        pltpu.sync_copy(oi_v, out_i.at[pl.ds(off, CHUNK)])
        return carry

    lax.fori_loop(0, N_CHUNKS, chunk_body, 0)


@jax.jit
def kernel(e1, r, ent_real, ent_img, rel_real, rel_img):
    er3 = ent_real.reshape(NUM_ENTITIES // 8, 8, EMBED_DIM)
    ei3 = ent_img.reshape(NUM_ENTITIES // 8, 8, EMBED_DIM)
    relpk = jnp.concatenate([rel_real, rel_img], axis=1)
    mesh = plsc.VectorSubcoreMesh(core_axis_name="c", subcore_axis_name="s")
    out_shape = jax.ShapeDtypeStruct((BATCH, EMBED_DIM), jnp.float32)
    fn = pl.kernel(
        _body,
        out_type=(out_shape, out_shape),
        mesh=mesh,
        scratch_types=[
            pltpu.VMEM((RPW,), jnp.int32),
            pltpu.VMEM((RPW,), jnp.int32),
            pltpu.VMEM((G, EMBED_DIM), jnp.float32),
            pltpu.VMEM((G, EMBED_DIM), jnp.float32),
            pltpu.VMEM((CHUNK, PK), jnp.float32),
            pltpu.VMEM((CHUNK, EMBED_DIM), jnp.float32),
            pltpu.VMEM((CHUNK, EMBED_DIM), jnp.float32),
            pltpu.SemaphoreType.DMA,
            pltpu.SemaphoreType.DMA,
        ],
        compiler_params=pltpu.CompilerParams(
            use_tc_tiling_on_sc=True, needs_layout_passes=False),
    )
    return fn(e1, r, er3, ei3, relpk)
